# baseline (speedup 1.0000x reference)
import jax
import jax.numpy as jnp
from jax.experimental import pallas as pl
from jax.experimental.pallas import tpu as pltpu

_SQRT_2_OVER_PI = 0.7978845608028654


def _make_body(NJ):
    def _mlp_body(x_ref, w1_ref, b1_ref, w2_ref, b2_ref, o_ref, p_ref):
        j = pl.program_id(1)

        @pl.when(j == 1)
        def _():
            o_ref[...] = p_ref[...] + b2_ref[...]

        @pl.when(j > 1)
        def _():
            o_ref[...] += p_ref[...]

        @pl.when(j < NJ)
        def _():
            h = jax.lax.dot_general(
                x_ref[...], w1_ref[...], (((1,), (1,)), ((), ())),
                preferred_element_type=jnp.float32)
            h = h + b1_ref[...]
            h = 0.5 * h * (1.0 + jnp.tanh(
                _SQRT_2_OVER_PI * (h + 0.044715 * h * h * h)))
            p_ref[...] = jax.lax.dot_general(
                h, w2_ref[...], (((1,), (1,)), ((), ())),
                preferred_element_type=jnp.float32)

    return _mlp_body


def kernel(x, W1, b1, W2, b2, Wg):
    B, S, D = x.shape
    M = B * S
    FF = W1.shape[0]
    xf = x.reshape(M, D)
    BM = min(1024, M)
    BF = min(512, FF)
    NJ = FF // BF
    grid = (M // BM, NJ + 1)
    clamp = NJ - 1
    out = pl.pallas_call(
        _make_body(NJ),
        grid=grid,
        in_specs=[
            pl.BlockSpec((BM, D), lambda i, j: (i, 0)),
            pl.BlockSpec((BF, D), lambda i, j: (jnp.minimum(j, clamp), 0)),
            pl.BlockSpec((1, BF), lambda i, j: (0, jnp.minimum(j, clamp))),
            pl.BlockSpec((D, BF), lambda i, j: (0, jnp.minimum(j, clamp))),
            pl.BlockSpec((1, D), lambda i, j: (0, 0)),
        ],
        out_specs=pl.BlockSpec((BM, D), lambda i, j: (i, 0)),
        out_shape=jax.ShapeDtypeStruct((M, D), jnp.float32),
        scratch_shapes=[pltpu.VMEM((BM, D), jnp.float32)],
        compiler_params=pltpu.CompilerParams(
            dimension_semantics=("parallel", "arbitrary"),
            vmem_limit_bytes=63 * 1024 * 1024,
        ),
    )(xf, W1, b1.reshape(1, FF), W2, b2.reshape(1, D))
    return out.reshape(B, S, D)


# speedup vs baseline: 1.0351x; 1.0351x over previous
import jax
import jax.numpy as jnp
from jax.experimental import pallas as pl
from jax.experimental.pallas import tpu as pltpu

_SQRT_2_OVER_PI = 0.7978845608028654


def _mlp_body(x_ref, w1_ref, b1_ref, w2_ref, b2_ref, o_ref):
    j = pl.program_id(1)
    h = jax.lax.dot_general(
        x_ref[...], w1_ref[...], (((1,), (1,)), ((), ())),
        preferred_element_type=jnp.float32)
    h = h + b1_ref[...]
    h = 0.5 * h * (1.0 + jnp.tanh(_SQRT_2_OVER_PI * (h + 0.044715 * h * h * h)))
    p = jax.lax.dot_general(
        h.astype(jnp.bfloat16), w2_ref[...], (((1,), (1,)), ((), ())),
        preferred_element_type=jnp.float32)

    @pl.when(j == 0)
    def _():
        o_ref[...] = p + b2_ref[...]

    @pl.when(j > 0)
    def _():
        o_ref[...] += p


def kernel(x, W1, b1, W2, b2, Wg):
    B, S, D = x.shape
    M = B * S
    FF = W1.shape[0]
    xf = x.reshape(M, D).astype(jnp.bfloat16)
    W1 = W1.astype(jnp.bfloat16)
    W2 = W2.astype(jnp.bfloat16)
    BM = min(1024, M)
    BF = min(1024, FF)
    grid = (M // BM, FF // BF)
    out = pl.pallas_call(
        _mlp_body,
        grid=grid,
        in_specs=[
            pl.BlockSpec((BM, D), lambda i, j: (i, 0)),
            pl.BlockSpec((BF, D), lambda i, j: (j, 0)),
            pl.BlockSpec((1, BF), lambda i, j: (0, j)),
            pl.BlockSpec((D, BF), lambda i, j: (0, j)),
            pl.BlockSpec((1, D), lambda i, j: (0, 0)),
        ],
        out_specs=pl.BlockSpec((BM, D), lambda i, j: (i, 0)),
        out_shape=jax.ShapeDtypeStruct((M, D), jnp.float32),
        compiler_params=pltpu.CompilerParams(
            dimension_semantics=("parallel", "arbitrary"),
            vmem_limit_bytes=63 * 1024 * 1024,
        ),
    )(xf, W1, b1.reshape(1, FF), W2, b2.reshape(1, D))
    return out.reshape(B, S, D)


# speedup vs baseline: 1.2086x; 1.1676x over previous
import jax
import jax.numpy as jnp
from jax.experimental import pallas as pl
from jax.experimental.pallas import tpu as pltpu

_SQRT_2_OVER_PI = 0.7978845608028654


def _mlp_body(x_ref, w1_ref, b1_ref, w2_ref, b2_ref, o_ref):
    j = pl.program_id(1)
    h = jax.lax.dot_general(
        x_ref[...], w1_ref[...], (((1,), (1,)), ((), ())),
        preferred_element_type=jnp.float32)
    h = h + b1_ref[...]
    h = 0.5 * h * (1.0 + jnp.tanh(_SQRT_2_OVER_PI * (h + 0.044715 * h * h * h)))
    D = o_ref.shape[1]
    NC = 4
    CD = D // NC
    first = j == 0
    for c in range(NC):
        sl = slice(c * CD, (c + 1) * CD)
        p = jax.lax.dot_general(
            h, w2_ref[sl, :], (((1,), (1,)), ((), ())),
            preferred_element_type=jnp.float32)
        base = jnp.where(
            first,
            jnp.broadcast_to(b2_ref[:, sl], p.shape),
            o_ref[:, sl])
        o_ref[:, sl] = base + p


def kernel(x, W1, b1, W2, b2, Wg):
    B, S, D = x.shape
    M = B * S
    FF = W1.shape[0]
    xf = x.reshape(M, D)
    BM = min(1024, M)
    BF = min(512, FF)
    grid = (M // BM, FF // BF)
    out = pl.pallas_call(
        _mlp_body,
        grid=grid,
        in_specs=[
            pl.BlockSpec((BM, D), lambda i, j: (i, 0)),
            pl.BlockSpec((BF, D), lambda i, j: (j, 0)),
            pl.BlockSpec((1, BF), lambda i, j: (0, j)),
            pl.BlockSpec((D, BF), lambda i, j: (0, j)),
            pl.BlockSpec((1, D), lambda i, j: (0, 0)),
        ],
        out_specs=pl.BlockSpec((BM, D), lambda i, j: (i, 0)),
        out_shape=jax.ShapeDtypeStruct((M, D), jnp.float32),
        compiler_params=pltpu.CompilerParams(
            dimension_semantics=("parallel", "arbitrary"),
            vmem_limit_bytes=63 * 1024 * 1024,
        ),
    )(xf, W1, b1.reshape(1, FF), W2, b2.reshape(1, D))
    return out.reshape(B, S, D)
